# baseline (speedup 1.0000x reference)
import functools

import jax
import jax.numpy as jnp
from jax import lax
from jax.experimental import pallas as pl
from jax.experimental.pallas import tpu as pltpu
from jax.experimental.pallas import tpu_sc as plsc

B = 4096 * 200
V = 100
D = 100
NW = 32
B_PER_W = B // NW
NC = 256
CH_OUT = NC * D
N_CH = B_PER_W // NC
L = 16
UNROLL = 8
INNER = CH_OUT // (L * UNROLL)


def _sc_gather(x_flat, W):
    mesh = plsc.VectorSubcoreMesh(core_axis_name="c", subcore_axis_name="s")

    @functools.partial(
        pl.kernel,
        mesh=mesh,
        out_type=jax.ShapeDtypeStruct((B * D,), jnp.float32),
        scratch_types=[
            pltpu.VMEM((V, D), jnp.float32),
            pltpu.VMEM((NC,), jnp.int32),
            pltpu.VMEM((NC,), jnp.int32),
            pltpu.VMEM((CH_OUT,), jnp.float32),
            pltpu.VMEM((CH_OUT,), jnp.float32),
            pltpu.SemaphoreType.DMA,
            pltpu.SemaphoreType.DMA,
            pltpu.SemaphoreType.DMA,
            pltpu.SemaphoreType.DMA,
        ],
        compiler_params=pltpu.CompilerParams(use_tc_tiling_on_sc=False,
                                             needs_layout_passes=False),
    )
    def k(x_hbm, w_hbm, out_hbm, w_v, xb0, xb1, st0, st1,
          s_x0, s_x1, s_w0, s_w1):
        xb = (xb0, xb1)
        st = (st0, st1)
        s_x = (s_x0, s_x1)
        s_w = (s_w0, s_w1)
        wid = lax.axis_index("s") * 2 + lax.axis_index("c")
        base = wid * B_PER_W
        obase = base * D

        def issue_x(ch, p):
            pltpu.async_copy(x_hbm.at[pl.ds(base + ch * NC, NC)],
                             xb[p], s_x[p])

        def wait_x(p):
            pltpu.make_async_copy(x_hbm.at[pl.ds(base, NC)],
                                  xb[p], s_x[p]).wait()

        def issue_write(ch, p):
            pltpu.async_copy(st[p],
                             out_hbm.at[pl.ds(obase + ch * CH_OUT, CH_OUT)],
                             s_w[p])

        def drain_write(p):
            pltpu.make_async_copy(st[p],
                                  out_hbm.at[pl.ds(obase, CH_OUT)],
                                  s_w[p]).wait()

        def compute(p):
            r0 = jnp.zeros((L,), jnp.int32)
            c0 = lax.iota(jnp.int32, L)

            def inner(t, carry):
                r, c = carry
                o = t * (L * UNROLL)
                for u in range(UNROLL):
                    xr = plsc.load_gather(xb[p], [r])
                    val = plsc.load_gather(w_v, [xr, c])
                    st[p][pl.ds(o + u * L, L)] = val
                    c2 = c + L
                    m = c2 >= D
                    c = jnp.where(m, c2 - D, c2)
                    r = jnp.where(m, r + 1, r)
                return r, c

            lax.fori_loop(0, INNER, inner, (r0, c0))

        def step(ch, p, do_prefetch, do_drain):
            wait_x(p)
            if do_prefetch:
                issue_x(ch + 1, 1 - p)
            if do_drain:
                drain_write(p)
            compute(p)
            issue_write(ch, p)

        pltpu.sync_copy(w_hbm, w_v)
        issue_x(0, 0)

        step(0, 0, True, False)
        step(1, 1, True, False)

        def pair_body(q, carry):
            step(q * 2, 0, True, True)
            step(q * 2 + 1, 1, True, True)
            return carry

        lax.fori_loop(1, N_CH // 2 - 1, pair_body, 0)

        step(N_CH - 2, 0, True, True)
        step(N_CH - 1, 1, False, True)

        drain_write(0)
        drain_write(1)

    return k(x_flat, W)


def kernel(x, W):
    out = _sc_gather(x.reshape(B), W)
    return out.reshape(4096, 200, D)


# speedup vs baseline: 1.6959x; 1.6959x over previous
import functools

import jax
import jax.numpy as jnp
from jax import lax
from jax.experimental import pallas as pl
from jax.experimental.pallas import tpu as pltpu
from jax.experimental.pallas import tpu_sc as plsc

B = 4096 * 200
V = 100
D = 100
NW = 32
B_PER_W = B // NW
CHUNK = 128
N_STEPS = B_PER_W // CHUNK


def _sc_gather(x_flat, W):
    mesh = plsc.VectorSubcoreMesh(core_axis_name="c", subcore_axis_name="s")

    @functools.partial(
        pl.kernel,
        mesh=mesh,
        out_type=jax.ShapeDtypeStruct((B, D), jnp.float32),
        scratch_types=[
            pltpu.VMEM((CHUNK,), jnp.int32),
            pltpu.VMEM((CHUNK, D), jnp.float32),
            pltpu.SemaphoreType.DMA,
            pltpu.SemaphoreType.DMA,
        ],
        compiler_params=pltpu.CompilerParams(use_tc_tiling_on_sc=False),
    )
    def k(x_hbm, w_hbm, out_hbm, idx_v, rows_v, s_g, s_i):
        wid = lax.axis_index("s") * 2 + lax.axis_index("c")
        base = wid * B_PER_W

        def issue_idx(g):
            pltpu.async_copy(x_hbm.at[pl.ds(base + g * CHUNK, CHUNK)],
                             idx_v, s_i)

        def wait_idx():
            pltpu.make_async_copy(x_hbm.at[pl.ds(base, CHUNK)],
                                  idx_v, s_i).wait()

        issue_idx(0)

        def body(g, carry):
            off = base + g * CHUNK
            wait_idx()
            pltpu.async_copy(w_hbm.at[idx_v], rows_v, s_g).wait()
            issue_idx(jnp.minimum(g + 1, N_STEPS - 1))
            pltpu.sync_copy(rows_v, out_hbm.at[pl.ds(off, CHUNK)])
            return carry

        lax.fori_loop(0, N_STEPS, body, 0)
        wait_idx()

    return k(x_flat, W)


def kernel(x, W):
    out = _sc_gather(x.reshape(B), W)
    return out.reshape(4096, 200, D)


# speedup vs baseline: 1.7532x; 1.0338x over previous
import functools

import jax
import jax.numpy as jnp
from jax import lax
from jax.experimental import pallas as pl
from jax.experimental.pallas import tpu as pltpu
from jax.experimental.pallas import tpu_sc as plsc

B = 4096 * 200
V = 100
D = 100
DP = 104
NW = 32
B_PER_W = B // NW
CHUNK = 128
N_STEPS = B_PER_W // CHUNK
N_PAIRS = N_STEPS // 2


def _sc_gather(x_flat, w_pad):
    mesh = plsc.VectorSubcoreMesh(core_axis_name="c", subcore_axis_name="s")

    @functools.partial(
        pl.kernel,
        mesh=mesh,
        out_type=jax.ShapeDtypeStruct((B, DP), jnp.float32),
        scratch_types=[
            pltpu.VMEM((CHUNK,), jnp.int32),
            pltpu.VMEM((CHUNK, DP), jnp.float32),
            pltpu.VMEM((CHUNK, DP), jnp.float32),
            pltpu.SemaphoreType.DMA,
            pltpu.SemaphoreType.DMA,
            pltpu.SemaphoreType.DMA,
        ],
        compiler_params=pltpu.CompilerParams(use_tc_tiling_on_sc=False),
    )
    def k(x_hbm, w_hbm, out_hbm, idx_v, rows0, rows1, s_g, s_o0, s_o1):
        rows = (rows0, rows1)
        s_o = (s_o0, s_o1)
        wid = lax.axis_index("s") * 2 + lax.axis_index("c")
        base = wid * B_PER_W

        def drain_write(g, p):
            pltpu.make_async_copy(
                rows[p], out_hbm.at[pl.ds(base + g * CHUNK, CHUNK)], s_o[p]
            ).wait()

        def step(g, p, do_drain):
            off = base + g * CHUNK
            pltpu.sync_copy(x_hbm.at[pl.ds(off, CHUNK)], idx_v)
            if do_drain:
                drain_write(g - 2, p)
            pltpu.async_copy(w_hbm.at[idx_v], rows[p], s_g).wait()
            pltpu.async_copy(rows[p], out_hbm.at[pl.ds(off, CHUNK)], s_o[p])

        step(0, 0, False)
        step(1, 1, False)

        def pair(q, carry):
            step(q * 2, 0, True)
            step(q * 2 + 1, 1, True)
            return carry

        lax.fori_loop(1, N_PAIRS, pair, 0)

        drain_write(N_STEPS - 2, 0)
        drain_write(N_STEPS - 1, 1)

    return k(x_flat, w_pad)


def kernel(x, W):
    w_pad = jnp.pad(W, ((0, 0), (0, DP - D)))
    out = _sc_gather(x.reshape(B), w_pad)
    return out[:, :D].reshape(4096, 200, D)


# speedup vs baseline: 2.7277x; 1.5559x over previous
import functools

import jax
import jax.numpy as jnp
from jax import lax
from jax.experimental import pallas as pl
from jax.experimental.pallas import tpu as pltpu
from jax.experimental.pallas import tpu_sc as plsc

B = 4096 * 200
V = 100
D = 100
DP = 104
NW = 32
B_PER_W = B // NW
CHUNK = 128
N_STEPS = B_PER_W // CHUNK
N_PAIRS = N_STEPS // 2


def _sc_gather(x_flat, w_pad):
    mesh = plsc.VectorSubcoreMesh(core_axis_name="c", subcore_axis_name="s")

    @functools.partial(
        pl.kernel,
        mesh=mesh,
        out_type=jax.ShapeDtypeStruct((B, DP), jnp.float32),
        scratch_types=[
            pltpu.VMEM((CHUNK,), jnp.int32),
            pltpu.VMEM_SHARED((V, DP), jnp.float32),
            pltpu.VMEM((CHUNK, DP), jnp.float32),
            pltpu.VMEM((CHUNK, DP), jnp.float32),
            pltpu.SemaphoreType.DMA,
            pltpu.SemaphoreType.DMA,
            pltpu.SemaphoreType.DMA,
        ],
        compiler_params=pltpu.CompilerParams(use_tc_tiling_on_sc=False),
    )
    def k(x_hbm, w_hbm, out_hbm, idx_v, w_sp, rows0, rows1, s_g, s_o0, s_o1):
        rows = (rows0, rows1)
        s_o = (s_o0, s_o1)
        sid = lax.axis_index("s")
        wid = sid * 2 + lax.axis_index("c")
        base = wid * B_PER_W

        @pl.when(sid == 0)
        def _():
            pltpu.sync_copy(w_hbm, w_sp)
        plsc.subcore_barrier()

        def drain_write(g, p):
            pltpu.make_async_copy(
                rows[p], out_hbm.at[pl.ds(base + g * CHUNK, CHUNK)], s_o[p]
            ).wait()

        def step(g, p, do_drain):
            off = base + g * CHUNK
            pltpu.sync_copy(x_hbm.at[pl.ds(off, CHUNK)], idx_v)
            if do_drain:
                drain_write(g - 2, p)
            pltpu.async_copy(w_sp.at[idx_v], rows[p], s_g).wait()
            pltpu.async_copy(rows[p], out_hbm.at[pl.ds(off, CHUNK)], s_o[p])

        step(0, 0, False)
        step(1, 1, False)

        def pair(q, carry):
            step(q * 2, 0, True)
            step(q * 2 + 1, 1, True)
            return carry

        lax.fori_loop(1, N_PAIRS, pair, 0)

        drain_write(N_STEPS - 2, 0)
        drain_write(N_STEPS - 1, 1)

    return k(x_flat, w_pad)


def kernel(x, W):
    w_pad = jnp.pad(W, ((0, 0), (0, DP - D)))
    out = _sc_gather(x.reshape(B), w_pad)
    return out[:, :D].reshape(4096, 200, D)
